# baseline (speedup 1.0000x reference)
import jax
import jax.numpy as jnp
from jax import lax
from jax.experimental import pallas as pl

_G = 4


def _fused_kernel(feats_ref, adj0_ref, adj1_ref, adj2_ref, adj3_ref,
                  w1_ref, b1_ref, wsp_ref, bd_ref, out_ref):
    adj_refs = (adj0_ref, adj1_ref, adj2_ref, adj3_ref)
    N = adj0_ref.shape[2]
    H = w1_ref.shape[1]
    f2d = feats_ref[...].reshape(_G * N, -1)
    xw = (jnp.dot(f2d.astype(jnp.bfloat16), w1_ref[...],
                  preferred_element_type=jnp.float32) + b1_ref[...])
    xwb = xw.astype(jnp.bfloat16)
    hs = []
    for g in range(_G):
        hq = []
        for r in range(4):
            a = adj_refs[r][g].astype(jnp.bfloat16)
            hr = jnp.maximum(
                jnp.dot(a, xwb[g * N:(g + 1) * N],
                        preferred_element_type=jnp.float32), 0)
            hq.append(hr.astype(jnp.bfloat16))
        hs.append(jnp.concatenate(hq, axis=0))
    hcat = jnp.concatenate(hs, axis=1)
    sp = jnp.dot(hcat, wsp_ref[...],
                 preferred_element_type=jnp.float32)
    s = sp[:, :_G]
    p = sp[:, _G:2 * _G] + sp[:, 2 * _G:]
    e = jnp.exp(jnp.tanh(s))
    numer = jnp.sum(e * p, axis=0, keepdims=True)
    den = jnp.sum(e, axis=0, keepdims=True)
    out_ref[0] = numer / den + bd_ref[...]


def kernel(feats, adj, W1, b1, att_w, Wd, bd):
    B, N, F = feats.shape
    H = W1.shape[1]
    bf = jnp.bfloat16
    wd = Wd[:, 0]
    wd_hi = wd.astype(bf).astype(jnp.float32)
    wd_lo = wd - wd_hi
    eye = jnp.eye(_G, dtype=jnp.float32)
    blk = jnp.concatenate([
        jnp.einsum('h,gk->ghk', att_w, eye),
        jnp.einsum('h,gk->ghk', wd_hi, eye),
        jnp.einsum('h,gk->ghk', wd_lo, eye),
    ], axis=2)
    wsp = blk.reshape(_G * H, 3 * _G).astype(bf)
    out = pl.pallas_call(
        _fused_kernel,
        grid=(B // _G,),
        in_specs=[
            pl.BlockSpec((_G, N, F), lambda b: (b, 0, 0)),
            pl.BlockSpec((_G, N // 4, N), lambda b: (b, 0, 0)),
            pl.BlockSpec((_G, N // 4, N), lambda b: (b, 1, 0)),
            pl.BlockSpec((_G, N // 4, N), lambda b: (b, 2, 0)),
            pl.BlockSpec((_G, N // 4, N), lambda b: (b, 3, 0)),
            pl.BlockSpec((F, H), lambda b: (0, 0)),
            pl.BlockSpec((1, H), lambda b: (0, 0)),
            pl.BlockSpec((_G * H, 3 * _G), lambda b: (0, 0)),
            pl.BlockSpec((1, 1), lambda b: (0, 0)),
        ],
        out_specs=pl.BlockSpec((1, 1, _G), lambda b: (b, 0, 0)),
        out_shape=jax.ShapeDtypeStruct((B // _G, 1, _G), jnp.float32),
    )(feats, adj, adj, adj, adj, W1.astype(bf), b1.reshape(1, H), wsp,
      bd.reshape(1, 1))
    return out.reshape(B)


# speedup vs baseline: 1.0072x; 1.0072x over previous
import jax
import jax.numpy as jnp
from jax import lax
from jax.experimental import pallas as pl

_G = 4
_R = 4


def _fused_kernel(feats_ref, adj0_ref, adj1_ref, adj2_ref, adj3_ref,
                  w1_ref, b1_ref, wspt_ref, bd_ref, out_ref):
    adj_refs = (adj0_ref, adj1_ref, adj2_ref, adj3_ref)
    N = adj0_ref.shape[2]
    H = w1_ref.shape[1]
    f2d = feats_ref[...].reshape(_G * N, -1)
    xw = (jnp.dot(f2d.astype(jnp.bfloat16), w1_ref[...],
                  preferred_element_type=jnp.float32) + b1_ref[...])
    xwb = xw.astype(jnp.bfloat16)
    hts = []
    for g in range(_G):
        xg = xwb[g * N:(g + 1) * N]
        cols = []
        for r in range(_R):
            a = adj_refs[r][g].astype(jnp.bfloat16)
            htr = lax.dot_general(xg, a, (((0,), (1,)), ((), ())),
                                  preferred_element_type=jnp.float32)
            cols.append(jnp.maximum(htr, 0).astype(jnp.bfloat16))
        hts.append(jnp.concatenate(cols, axis=1))
    hcatt = jnp.concatenate(hts, axis=0)
    spt = jnp.dot(wspt_ref[...], hcatt,
                  preferred_element_type=jnp.float32)
    s = spt[:_G]
    p = spt[_G:2 * _G] + spt[2 * _G:]
    e = jnp.exp(jnp.tanh(s))
    numer = jnp.sum(e * p, axis=1, keepdims=True)
    den = jnp.sum(e, axis=1, keepdims=True)
    out_ref[0] = numer / den + bd_ref[...]


def kernel(feats, adj, W1, b1, att_w, Wd, bd):
    B, N, F = feats.shape
    H = W1.shape[1]
    bf = jnp.bfloat16
    wd = Wd[:, 0]
    wd_hi = wd.astype(bf).astype(jnp.float32)
    wd_lo = wd - wd_hi
    eye = jnp.eye(_G, dtype=jnp.float32)
    blk = jnp.concatenate([
        jnp.einsum('h,kg->kgh', att_w, eye),
        jnp.einsum('h,kg->kgh', wd_hi, eye),
        jnp.einsum('h,kg->kgh', wd_lo, eye),
    ], axis=0)
    wspt = blk.reshape(3 * _G, _G * H).astype(bf)
    out = pl.pallas_call(
        _fused_kernel,
        grid=(B // _G,),
        in_specs=[
            pl.BlockSpec((_G, N, F), lambda b: (b, 0, 0)),
            pl.BlockSpec((_G, N // _R, N), lambda b: (b, 0, 0)),
            pl.BlockSpec((_G, N // _R, N), lambda b: (b, 1, 0)),
            pl.BlockSpec((_G, N // _R, N), lambda b: (b, 2, 0)),
            pl.BlockSpec((_G, N // _R, N), lambda b: (b, 3, 0)),
            pl.BlockSpec((F, H), lambda b: (0, 0)),
            pl.BlockSpec((1, H), lambda b: (0, 0)),
            pl.BlockSpec((3 * _G, _G * H), lambda b: (0, 0)),
            pl.BlockSpec((1, 1), lambda b: (0, 0)),
        ],
        out_specs=pl.BlockSpec((1, _G, 1), lambda b: (b, 0, 0)),
        out_shape=jax.ShapeDtypeStruct((B // _G, _G, 1), jnp.float32),
    )(feats, adj, adj, adj, adj, W1.astype(bf), b1.reshape(1, H), wspt,
      bd.reshape(1, 1))
    return out.reshape(B)


# speedup vs baseline: 1.0076x; 1.0003x over previous
import jax
import jax.numpy as jnp
from jax import lax
from jax.experimental import pallas as pl

_G = 4
_R = 4


def _fused_kernel(feats_ref, adj0_ref, adj1_ref, adj2_ref, adj3_ref,
                  w1_ref, b1_ref, wspt_ref, bd_ref, out_ref):
    adj_refs = (adj0_ref, adj1_ref, adj2_ref, adj3_ref)
    N = adj0_ref.shape[2]
    H = w1_ref.shape[1]
    f2d = feats_ref[...].reshape(_G * N, -1)
    xw = (jnp.dot(f2d.astype(jnp.bfloat16), w1_ref[...],
                  preferred_element_type=jnp.float32) + b1_ref[...])
    xwb = xw.astype(jnp.bfloat16)
    hts = []
    for g in range(_G):
        xg = xwb[g * N:(g + 1) * N]
        a = jnp.concatenate(
            [adj_refs[r][g] for r in range(_R)],
            axis=0).astype(jnp.bfloat16)
        ht = lax.dot_general(xg, a, (((0,), (1,)), ((), ())),
                             preferred_element_type=jnp.float32)
        hts.append(jnp.maximum(ht, 0).astype(jnp.bfloat16))
    hcatt = jnp.concatenate(hts, axis=0)
    spt = jnp.dot(wspt_ref[...], hcatt,
                  preferred_element_type=jnp.float32)
    s = spt[:_G]
    p = spt[_G:2 * _G] + spt[2 * _G:]
    e = jnp.exp(jnp.tanh(s))
    numer = jnp.sum(e * p, axis=1, keepdims=True)
    den = jnp.sum(e, axis=1, keepdims=True)
    out_ref[0] = numer / den + bd_ref[...]


def kernel(feats, adj, W1, b1, att_w, Wd, bd):
    B, N, F = feats.shape
    H = W1.shape[1]
    bf = jnp.bfloat16
    wd = Wd[:, 0]
    wd_hi = wd.astype(bf).astype(jnp.float32)
    wd_lo = wd - wd_hi
    eye = jnp.eye(_G, dtype=jnp.float32)
    blk = jnp.concatenate([
        jnp.einsum('h,kg->kgh', att_w, eye),
        jnp.einsum('h,kg->kgh', wd_hi, eye),
        jnp.einsum('h,kg->kgh', wd_lo, eye),
    ], axis=0)
    wspt = blk.reshape(3 * _G, _G * H).astype(bf)
    out = pl.pallas_call(
        _fused_kernel,
        grid=(B // _G,),
        in_specs=[
            pl.BlockSpec((_G, N, F), lambda b: (b, 0, 0)),
            pl.BlockSpec((_G, N // _R, N), lambda b: (b, 0, 0)),
            pl.BlockSpec((_G, N // _R, N), lambda b: (b, 1, 0)),
            pl.BlockSpec((_G, N // _R, N), lambda b: (b, 2, 0)),
            pl.BlockSpec((_G, N // _R, N), lambda b: (b, 3, 0)),
            pl.BlockSpec((F, H), lambda b: (0, 0)),
            pl.BlockSpec((1, H), lambda b: (0, 0)),
            pl.BlockSpec((3 * _G, _G * H), lambda b: (0, 0)),
            pl.BlockSpec((1, 1), lambda b: (0, 0)),
        ],
        out_specs=pl.BlockSpec((1, _G, 1), lambda b: (b, 0, 0)),
        out_shape=jax.ShapeDtypeStruct((B // _G, _G, 1), jnp.float32),
    )(feats, adj, adj, adj, adj, W1.astype(bf), b1.reshape(1, H), wspt,
      bd.reshape(1, 1))
    return out.reshape(B)


# speedup vs baseline: 1.1198x; 1.1114x over previous
import jax
import jax.numpy as jnp
from jax import lax
from jax.experimental import pallas as pl

_G = 8
_R = 4


def _fused_kernel(feats_ref, adj0_ref, adj1_ref, adj2_ref, adj3_ref,
                  w1_ref, b1_ref, wspt_ref, bd_ref, out_ref):
    adj_refs = (adj0_ref, adj1_ref, adj2_ref, adj3_ref)
    N = adj0_ref.shape[2]
    H = w1_ref.shape[1]
    f2d = feats_ref[...].reshape(_G * N, -1)
    xw = (jnp.dot(f2d.astype(jnp.bfloat16), w1_ref[...],
                  preferred_element_type=jnp.float32) + b1_ref[...])
    xwb = xw.astype(jnp.bfloat16)
    hts = []
    for g in range(_G):
        xg = xwb[g * N:(g + 1) * N]
        a = jnp.concatenate(
            [adj_refs[r][g] for r in range(_R)],
            axis=0).astype(jnp.bfloat16)
        ht = lax.dot_general(xg, a, (((0,), (1,)), ((), ())),
                             preferred_element_type=jnp.float32)
        hts.append(jnp.maximum(ht, 0).astype(jnp.bfloat16))
    hcatt = jnp.concatenate(hts, axis=0)
    spt = jnp.dot(wspt_ref[...], hcatt,
                  preferred_element_type=jnp.float32)
    s = spt[:_G]
    p = spt[_G:2 * _G] + spt[2 * _G:]
    e = jnp.exp(jnp.tanh(s))
    numer = jnp.sum(e * p, axis=1, keepdims=True)
    den = jnp.sum(e, axis=1, keepdims=True)
    out_ref[0] = numer / den + bd_ref[...]


def kernel(feats, adj, W1, b1, att_w, Wd, bd):
    B, N, F = feats.shape
    H = W1.shape[1]
    bf = jnp.bfloat16
    wd = Wd[:, 0]
    wd_hi = wd.astype(bf).astype(jnp.float32)
    wd_lo = wd - wd_hi
    eye = jnp.eye(_G, dtype=jnp.float32)
    blk = jnp.concatenate([
        jnp.einsum('h,kg->kgh', att_w, eye),
        jnp.einsum('h,kg->kgh', wd_hi, eye),
        jnp.einsum('h,kg->kgh', wd_lo, eye),
    ], axis=0)
    wspt = blk.reshape(3 * _G, _G * H).astype(bf)
    out = pl.pallas_call(
        _fused_kernel,
        grid=(B // _G,),
        in_specs=[
            pl.BlockSpec((_G, N, F), lambda b: (b, 0, 0)),
            pl.BlockSpec((_G, N // _R, N), lambda b: (b, 0, 0)),
            pl.BlockSpec((_G, N // _R, N), lambda b: (b, 1, 0)),
            pl.BlockSpec((_G, N // _R, N), lambda b: (b, 2, 0)),
            pl.BlockSpec((_G, N // _R, N), lambda b: (b, 3, 0)),
            pl.BlockSpec((F, H), lambda b: (0, 0)),
            pl.BlockSpec((1, H), lambda b: (0, 0)),
            pl.BlockSpec((3 * _G, _G * H), lambda b: (0, 0)),
            pl.BlockSpec((1, 1), lambda b: (0, 0)),
        ],
        out_specs=pl.BlockSpec((1, _G, 1), lambda b: (b, 0, 0)),
        out_shape=jax.ShapeDtypeStruct((B // _G, _G, 1), jnp.float32),
    )(feats, adj, adj, adj, adj, W1.astype(bf), b1.reshape(1, H), wspt,
      bd.reshape(1, 1))
    return out.reshape(B)
